# baseline (speedup 1.0000x reference)
import numpy as np
import jax
import jax.numpy as jnp
from jax import lax
from jax.experimental import pallas as pl
from jax.experimental.pallas import tpu as pltpu

_BS = 64


def _h64_np():
    H = np.array([[1.0]], dtype=np.float32)
    for _ in range(6):
        H = np.kron(H, np.array([[1.0, 1.0], [1.0, -1.0]], dtype=np.float32))
    return H / np.sqrt(_BS)


def _ternary_fwd(w, s_tilde, t, delta):
    s = jax.nn.softplus(s_tilde)
    diff = w - t[:, None]
    q = jnp.sign(diff) * (jnp.abs(diff) > delta[:, None]).astype(w.dtype)
    return s[:, None] * q


def _softplus(x):
    return jnp.maximum(x, 0.0) + jnp.log1p(jnp.exp(-jnp.abs(x)))


def _sigmoid(x):
    return 1.0 / (1.0 + jnp.exp(-x))


def _mamba_kernel_body(B, TL, N, K):
    def body(x_ref, M_ref, WinT_ref, cwT_ref, cb_ref, rin_ref, sin_ref,
             rc_ref, sc_ref, xp_ref, dtw_ref, dtb_ref, AT2_ref, dsk_ref,
             WoT_ref, out_ref,
             tail_ref, uin_scr, h_ref, u_scr, z_scr, dt_scr, g_scr, y_scr,
             bb_scr, cc_scr):
        j = pl.program_id(0)

        @pl.when(j == 0)
        def _():
            tail_ref[...] = jnp.zeros_like(tail_ref)
            h_ref[...] = jnp.zeros_like(h_ref)

        DI = cb_ref.shape[1]
        R = dtw_ref.shape[1]
        D = M_ref.shape[0]
        BT_rows = B * TL

        xb = x_ref[...].reshape(BT_rows, D)
        xm = jnp.dot(xb.astype(jnp.bfloat16), M_ref[...].astype(jnp.bfloat16),
                     preferred_element_type=jnp.float32)
        xq = jnp.clip(jnp.round(xm * rin_ref[...]), -7.0, 7.0) * sin_ref[...]
        xz = jnp.dot(xq.astype(jnp.bfloat16), WinT_ref[...].astype(jnp.bfloat16),
                     preferred_element_type=jnp.float32)
        u_c = xz[:, :DI]
        z_scr[...] = xz[:, DI:]

        uin_scr[...] = u_c.astype(jnp.bfloat16)
        cw = cwT_ref[...]
        uc_rows = []
        for b in range(B):
            ub = uin_scr[b * TL:(b + 1) * TL]
            tb = tail_ref[b * 8:(b + 1) * 8]
            ext = jnp.concatenate([tb, ub], axis=0).astype(jnp.float32)
            ucv = cw[K - 1:K, :] * ext[8:8 + TL]
            for jj in range(1, K):
                ucv = ucv + cw[K - 1 - jj:K - jj, :] * ext[8 - jj:8 - jj + TL]
            uc_rows.append(ucv + cb_ref[...])
            tail_ref[b * 8:(b + 1) * 8] = uin_scr[(b + 1) * TL - 8:(b + 1) * TL]
        ucv_all = jnp.concatenate(uc_rows, axis=0)

        ua = ucv_all * _sigmoid(ucv_all)
        u_ssm = jnp.clip(jnp.round(ua * rc_ref[...]), -7.0, 7.0) * sc_ref[...]
        u_scr[...] = u_ssm

        xw_dt = xp_ref[:R]
        B_w = xp_ref[R:R + N].astype(jnp.bfloat16)
        C_w = xp_ref[R + N:R + 2 * N].astype(jnp.bfloat16)
        u_bf = u_ssm.astype(jnp.bfloat16)
        dtR = lax.dot_general(u_bf, xw_dt.astype(jnp.bfloat16),
                              (((1,), (1,)), ((), ())),
                              preferred_element_type=jnp.float32)
        dtlog = lax.dot_general(dtR.astype(jnp.bfloat16),
                                dtw_ref[...].astype(jnp.bfloat16),
                                (((1,), (1,)), ((), ())),
                                preferred_element_type=jnp.float32)
        dt_c = _softplus(dtlog + dtb_ref[...])
        dt_scr[...] = dt_c
        g_scr[...] = dt_c * u_ssm
        LREP = DI // 128
        for b in range(B):
            ub_bf = u_bf[b * TL:(b + 1) * TL]
            Bm = lax.dot_general(ub_bf, B_w, (((1,), (1,)), ((), ())),
                                 preferred_element_type=jnp.float32)
            Cm = lax.dot_general(ub_bf, C_w, (((1,), (1,)), ((), ())),
                                 preferred_element_type=jnp.float32)
            bb_scr[b] = jnp.broadcast_to(Bm[:, :, None], (TL, N, 128))
            cc_scr[b] = jnp.broadcast_to(Cm[:, :, None], (TL, N, 128))

        AT2 = AT2_ref[...]
        hs = [h_ref[b * N:(b + 1) * N] for b in range(B)]
        for t in range(TL):
            for b in range(B):
                r = b * TL + t
                dt_row = dt_scr[r:r + 1, :]
                a = jnp.exp2(AT2 * dt_row)
                g_row = g_scr[r:r + 1, :]
                bful = pltpu.repeat(bb_scr[b, t], LREP, axis=1)
                cful = pltpu.repeat(cc_scr[b, t], LREP, axis=1)
                hs[b] = a * hs[b] + bful * g_row
                y_scr[r:r + 1, :] = jnp.sum(cful * hs[b], axis=0, keepdims=True)
        for b in range(B):
            h_ref[b * N:(b + 1) * N] = hs[b]

        z = z_scr[...]
        y = (y_scr[...] + u_scr[...] * dsk_ref[...]) * (z * _sigmoid(z))
        out_ref[...] = jnp.dot(y.astype(jnp.bfloat16),
                             WoT_ref[...].astype(jnp.bfloat16),
                             preferred_element_type=jnp.float32
                             ).reshape(B, TL, D)

    return body


def kernel(x, sign1, sign2, perm, pvals_in, pvals_conv, in_proj_w, s_in,
           t_in, delta_in, conv_w, conv_b, x_proj_w, dt_proj_w, dt_proj_b,
           A_log, D_skip, out_proj_w, s_out, t_out, delta_out):
    B, L, D = x.shape
    DI, K = conv_w.shape
    N = A_log.shape[1]
    R = dt_proj_w.shape[1]
    TL = 32
    NC = L // TL

    BD = jnp.asarray(np.kron(np.eye(D // _BS, dtype=np.float32), _h64_np()))
    M = (BD * sign1[None, :])[:, perm] * sign2[None, :]
    WinT = _ternary_fwd(in_proj_w, s_in, t_in, delta_in).T
    WoT = _ternary_fwd(out_proj_w, s_out, t_out, delta_out).T
    AT2 = ((-jnp.exp(A_log)) * np.float32(np.log2(np.e))).T
    cwT = conv_w.T.astype(jnp.bfloat16).astype(jnp.float32)
    scale_in = pvals_in / 7.0
    scale_c = pvals_conv / 7.0
    rin = (1.0 / (scale_in + 1e-8)).reshape(1, D)
    sin = scale_in.reshape(1, D)
    rc = (1.0 / (scale_c + 1e-8)).reshape(1, DI)
    sc = scale_c.reshape(1, DI)
    cb = conv_b.reshape(1, DI)
    dtb = dt_proj_b.reshape(1, DI)
    dsk = D_skip.reshape(1, DI)

    full = lambda shp: pl.BlockSpec(shp, lambda j, _s=None: (0,) * len(shp))
    out = pl.pallas_call(
        _mamba_kernel_body(B, TL, N, K),
        out_shape=jax.ShapeDtypeStruct((B, L, D), jnp.float32),
        grid=(NC,),
        in_specs=[
            pl.BlockSpec((B, TL, D), lambda j: (0, j, 0)),
            full((D, D)),
            full((D, 2 * DI)),
            full((K, DI)),
            full((1, DI)),
            full((1, D)),
            full((1, D)),
            full((1, DI)),
            full((1, DI)),
            full((R + 2 * N, DI)),
            full((DI, R)),
            full((1, DI)),
            full((N, DI)),
            full((1, DI)),
            full((DI, D)),
        ],
        out_specs=pl.BlockSpec((B, TL, D), lambda j: (0, j, 0)),
        scratch_shapes=[
            pltpu.VMEM((B * 8, DI), jnp.bfloat16),
            pltpu.VMEM((B * TL, DI), jnp.bfloat16),
            pltpu.VMEM((B * N, DI), jnp.float32),
            pltpu.VMEM((B * TL, DI), jnp.float32),
            pltpu.VMEM((B * TL, DI), jnp.float32),
            pltpu.VMEM((B * TL, DI), jnp.float32),
            pltpu.VMEM((B * TL, DI), jnp.float32),
            pltpu.VMEM((B * TL, DI), jnp.float32),
            pltpu.VMEM((B, TL, N, 128), jnp.float32),
            pltpu.VMEM((B, TL, N, 128), jnp.float32),
        ],
        compiler_params=pltpu.CompilerParams(
            dimension_semantics=("arbitrary",),
            vmem_limit_bytes=48 * 1024 * 1024,
        ),
        name="quant_mamba_fused",
    )(x, M, WinT, cwT, cb, rin, sin, rc, sc, x_proj_w, dt_proj_w, dtb, AT2,
      dsk, WoT)
    return out


# speedup vs baseline: 1.0637x; 1.0637x over previous
import numpy as np
import jax
import jax.numpy as jnp
from jax import lax
from jax.experimental import pallas as pl
from jax.experimental.pallas import tpu as pltpu

_BS = 64


def _h64_np():
    H = np.array([[1.0]], dtype=np.float32)
    for _ in range(6):
        H = np.kron(H, np.array([[1.0, 1.0], [1.0, -1.0]], dtype=np.float32))
    return H / np.sqrt(_BS)


def _ternary_fwd(w, s_tilde, t, delta):
    s = jax.nn.softplus(s_tilde)
    diff = w - t[:, None]
    q = jnp.sign(diff) * (jnp.abs(diff) > delta[:, None]).astype(w.dtype)
    return s[:, None] * q


def _softplus(x):
    return jnp.maximum(x, 0.0) + jnp.log1p(jnp.exp(-jnp.abs(x)))


def _sigmoid(x):
    return 1.0 / (1.0 + jnp.exp(-x))


def _mamba_kernel_body(B, TL, N, K):
    def body(x_ref, M_ref, WinT_ref, cwT_ref, cb_ref, rin_ref, sin_ref,
             rc_ref, sc_ref, xp_ref, dtw_ref, dtb_ref, AT2_ref, dsk_ref,
             WoT_ref, out_ref,
             tail_ref, uin_scr, h_ref, u_scr, z_scr, dt_scr, g_scr, y_scr,
             bb_scr, cc_scr):
        j = pl.program_id(0)

        @pl.when(j == 0)
        def _():
            tail_ref[...] = jnp.zeros_like(tail_ref)
            h_ref[...] = jnp.zeros_like(h_ref)

        DI = cb_ref.shape[1]
        R = dtw_ref.shape[1]
        D = M_ref.shape[0]
        BT_rows = B * TL

        xb = x_ref[...].reshape(BT_rows, D)
        xm = jnp.dot(xb.astype(jnp.bfloat16), M_ref[...].astype(jnp.bfloat16),
                     preferred_element_type=jnp.float32)
        xq = jnp.clip(jnp.round(xm * rin_ref[...]), -7.0, 7.0) * sin_ref[...]
        xz = jnp.dot(xq.astype(jnp.bfloat16), WinT_ref[...].astype(jnp.bfloat16),
                     preferred_element_type=jnp.float32)
        u_c = xz[:, :DI]
        z_scr[...] = xz[:, DI:]

        uin_scr[...] = u_c.astype(jnp.bfloat16)
        cw = cwT_ref[...]
        uc_rows = []
        for b in range(B):
            ub = uin_scr[b * TL:(b + 1) * TL]
            tb = tail_ref[b * 8:(b + 1) * 8]
            ext = jnp.concatenate([tb, ub], axis=0).astype(jnp.float32)
            ucv = cw[K - 1:K, :] * ext[8:8 + TL]
            for jj in range(1, K):
                ucv = ucv + cw[K - 1 - jj:K - jj, :] * ext[8 - jj:8 - jj + TL]
            uc_rows.append(ucv + cb_ref[...])
            tail_ref[b * 8:(b + 1) * 8] = uin_scr[(b + 1) * TL - 8:(b + 1) * TL]
        ucv_all = jnp.concatenate(uc_rows, axis=0)

        ua = ucv_all * _sigmoid(ucv_all)
        u_ssm = jnp.clip(jnp.round(ua * rc_ref[...]), -7.0, 7.0) * sc_ref[...]
        u_scr[...] = u_ssm

        xw_dt = xp_ref[:R]
        B_w = xp_ref[R:R + N].astype(jnp.bfloat16)
        C_w = xp_ref[R + N:R + 2 * N].astype(jnp.bfloat16)
        u_bf = u_ssm.astype(jnp.bfloat16)
        dtR = lax.dot_general(u_bf, xw_dt.astype(jnp.bfloat16),
                              (((1,), (1,)), ((), ())),
                              preferred_element_type=jnp.float32)
        dtlog = lax.dot_general(dtR.astype(jnp.bfloat16),
                                dtw_ref[...].astype(jnp.bfloat16),
                                (((1,), (1,)), ((), ())),
                                preferred_element_type=jnp.float32)
        dt_c = _softplus(dtlog + dtb_ref[...])
        dt_scr[...] = dt_c
        g_scr[...] = dt_c * u_ssm
        LREP = DI // 128
        for b in range(B):
            ub_bf = u_bf[b * TL:(b + 1) * TL]
            Bm = lax.dot_general(ub_bf, B_w, (((1,), (1,)), ((), ())),
                                 preferred_element_type=jnp.float32)
            Cm = lax.dot_general(ub_bf, C_w, (((1,), (1,)), ((), ())),
                                 preferred_element_type=jnp.float32)
            bb_scr[b] = jnp.broadcast_to(Bm[:, :, None], (TL, N, 128))
            cc_scr[b] = jnp.broadcast_to(Cm[:, :, None], (TL, N, 128))

        AT2 = AT2_ref[...]
        hs = [h_ref[b * N:(b + 1) * N] for b in range(B)]
        for t in range(TL):
            for b in range(B):
                r = b * TL + t
                dt_row = dt_scr[r:r + 1, :]
                a = jnp.exp2(AT2 * dt_row)
                g_row = g_scr[r:r + 1, :]
                bful = pltpu.repeat(bb_scr[b, t], LREP, axis=1)
                cful = pltpu.repeat(cc_scr[b, t], LREP, axis=1)
                hs[b] = a * hs[b] + bful * g_row
                y_scr[r:r + 1, :] = jnp.sum(cful * hs[b], axis=0, keepdims=True)
        for b in range(B):
            h_ref[b * N:(b + 1) * N] = hs[b]

        z = z_scr[...]
        y = (y_scr[...] + u_scr[...] * dsk_ref[...]) * (z * _sigmoid(z))
        out_ref[...] = jnp.dot(y.astype(jnp.bfloat16),
                             WoT_ref[...].astype(jnp.bfloat16),
                             preferred_element_type=jnp.float32
                             ).reshape(B, TL, D)

    return body


def kernel(x, sign1, sign2, perm, pvals_in, pvals_conv, in_proj_w, s_in,
           t_in, delta_in, conv_w, conv_b, x_proj_w, dt_proj_w, dt_proj_b,
           A_log, D_skip, out_proj_w, s_out, t_out, delta_out):
    B, L, D = x.shape
    DI, K = conv_w.shape
    N = A_log.shape[1]
    R = dt_proj_w.shape[1]
    TL = 64
    NC = L // TL

    BD = jnp.asarray(np.kron(np.eye(D // _BS, dtype=np.float32), _h64_np()))
    M = (BD * sign1[None, :])[:, perm] * sign2[None, :]
    WinT = _ternary_fwd(in_proj_w, s_in, t_in, delta_in).T
    WoT = _ternary_fwd(out_proj_w, s_out, t_out, delta_out).T
    AT2 = ((-jnp.exp(A_log)) * np.float32(np.log2(np.e))).T
    cwT = conv_w.T.astype(jnp.bfloat16).astype(jnp.float32)
    scale_in = pvals_in / 7.0
    scale_c = pvals_conv / 7.0
    rin = (1.0 / (scale_in + 1e-8)).reshape(1, D)
    sin = scale_in.reshape(1, D)
    rc = (1.0 / (scale_c + 1e-8)).reshape(1, DI)
    sc = scale_c.reshape(1, DI)
    cb = conv_b.reshape(1, DI)
    dtb = dt_proj_b.reshape(1, DI)
    dsk = D_skip.reshape(1, DI)

    full = lambda shp: pl.BlockSpec(shp, lambda j, _s=None: (0,) * len(shp))
    out = pl.pallas_call(
        _mamba_kernel_body(B, TL, N, K),
        out_shape=jax.ShapeDtypeStruct((B, L, D), jnp.float32),
        grid=(NC,),
        in_specs=[
            pl.BlockSpec((B, TL, D), lambda j: (0, j, 0)),
            full((D, D)),
            full((D, 2 * DI)),
            full((K, DI)),
            full((1, DI)),
            full((1, D)),
            full((1, D)),
            full((1, DI)),
            full((1, DI)),
            full((R + 2 * N, DI)),
            full((DI, R)),
            full((1, DI)),
            full((N, DI)),
            full((1, DI)),
            full((DI, D)),
        ],
        out_specs=pl.BlockSpec((B, TL, D), lambda j: (0, j, 0)),
        scratch_shapes=[
            pltpu.VMEM((B * 8, DI), jnp.bfloat16),
            pltpu.VMEM((B * TL, DI), jnp.bfloat16),
            pltpu.VMEM((B * N, DI), jnp.float32),
            pltpu.VMEM((B * TL, DI), jnp.float32),
            pltpu.VMEM((B * TL, DI), jnp.float32),
            pltpu.VMEM((B * TL, DI), jnp.float32),
            pltpu.VMEM((B * TL, DI), jnp.float32),
            pltpu.VMEM((B * TL, DI), jnp.float32),
            pltpu.VMEM((B, TL, N, 128), jnp.float32),
            pltpu.VMEM((B, TL, N, 128), jnp.float32),
        ],
        compiler_params=pltpu.CompilerParams(
            dimension_semantics=("arbitrary",),
            vmem_limit_bytes=56 * 1024 * 1024,
        ),
        name="quant_mamba_fused",
    )(x, M, WinT, cwT, cb, rin, sin, rc, sc, x_proj_w, dt_proj_w, dtb, AT2,
      dsk, WoT)
    return out
